# baseline (speedup 1.0000x reference)
import functools

import jax
import jax.numpy as jnp
from jax import lax
from jax.experimental import pallas as pl
from jax.experimental.pallas import tpu as pltpu
from jax.experimental.pallas import tpu_sc as plsc

N = 10000
NP = 10240
E = 160000
U = 64
B = 4
FB = 80
NSC = 2
NT = 16
CPB = 2 * FB
LGB = CPB // 32
K = 160
EPT = 10240
NCH = EPT // K
RPT = NP // NT
NWO = RPT // K
NBUF = 4
NBLK = 2000


def _spmm_body(x_hbm, edge_hbm, vs_hbm, out1_hbm, out2_hbm, acc, *bufs):
    ebufs = bufs[0:NBUF]
    vbufs = bufs[NBUF:2 * NBUF]
    rowss = bufs[2 * NBUF:3 * NBUF]
    esems = bufs[3 * NBUF:4 * NBUF]
    vsems = bufs[4 * NBUF:5 * NBUF]
    gsems = bufs[5 * NBUF:6 * NBUF]
    ssems = bufs[6 * NBUF:7 * NBUF]
    cid = lax.axis_index("c")
    sid = lax.axis_index("s")
    wid = cid * NT + sid

    def ecopy_start(ch, s):
        pltpu.async_copy(edge_hbm.at[wid, ch], ebufs[s], esems[s])
        pltpu.async_copy(vs_hbm.at[sid, ch], vbufs[s], vsems[s])

    def ecopy_wait(ch, s):
        pltpu.make_async_copy(edge_hbm.at[wid, ch], ebufs[s],
                              esems[s]).wait()
        pltpu.make_async_copy(vs_hbm.at[sid, ch], vbufs[s],
                              vsems[s]).wait()

    def scatter_start(s):
        pltpu.async_copy(rowss[s], acc.at[ebufs[s].at[1]], ssems[s],
                         add=True)

    def scatter_wait(s):
        pltpu.make_async_copy(rowss[s], acc.at[ebufs[s].at[1]],
                              ssems[s]).wait()

    def scale(s):
        rb = rowss[s]
        vb = vbufs[s]

        @plsc.parallel_loop(0, K, unroll=4)
        def _(e):
            vvb = vb[e, pl.ds(0, 32)]
            for j in range(LGB):
                sl = pl.ds(j * 32, 32)
                rb[e, sl] = rb[e, sl] * vvb

    def zero_own_slice():
        zero32 = jnp.zeros((32,), jnp.bfloat16)

        def zrow(i, carry):
            for j in range(LGB):
                rowss[0][i, pl.ds(j * 32, 32)] = zero32
            return carry

        lax.fori_loop(0, K, zrow, 0)
        for kk in range(NWO):
            pltpu.sync_copy(rowss[0], acc.at[pl.ds(sid * RPT + kk * K, K)])

    def hop(src_hbm):
        def gather_start(s):
            pltpu.async_copy(src_hbm.at[ebufs[s].at[0]], rowss[s], gsems[s])

        def gather_wait(s):
            pltpu.make_async_copy(src_hbm.at[ebufs[s].at[0]], rowss[s],
                                  gsems[s]).wait()

        ecopy_start(0, 0)
        ecopy_start(1, 1)
        ecopy_wait(0, 0)
        gather_start(0)

        def chunk_iter(ch, t):
            @pl.when(ch >= 2)
            def _():
                scatter_wait((t + 2) % NBUF)

            @pl.when(ch + 2 < NCH)
            def _():
                ecopy_start(ch + 2, (t + 2) % NBUF)

            @pl.when(ch + 1 < NCH)
            def _():
                ecopy_wait(ch + 1, (t + 1) % NBUF)
                gather_start((t + 1) % NBUF)

            gather_wait(t)
            scale(t)
            scatter_start(t)

        def quad(p, carry):
            for t in range(NBUF):
                chunk_iter(NBUF * p + t, t)
            return carry

        lax.fori_loop(0, NCH // NBUF, quad, 0)
        scatter_wait((NCH - 2) % NBUF)
        scatter_wait((NCH - 1) % NBUF)

    def writeout(dst_hbm):
        for kk in range(NWO):
            r0 = sid * RPT + kk * K
            pltpu.sync_copy(acc.at[pl.ds(r0, K)], rowss[kk])
            pltpu.async_copy(rowss[kk],
                             dst_hbm.at[pl.ds(cid * NP + r0, K)],
                             gsems[kk])
        for kk in range(NWO):
            r0 = sid * RPT + kk * K
            pltpu.make_async_copy(
                rowss[kk], dst_hbm.at[pl.ds(cid * NP + r0, K)],
                gsems[kk]).wait()

    zero_own_slice()
    plsc.subcore_barrier()
    hop(x_hbm)
    plsc.subcore_barrier()
    writeout(out1_hbm)
    zero_own_slice()
    plsc.subcore_barrier()
    hop(out1_hbm)
    plsc.subcore_barrier()
    writeout(out2_hbm)


@functools.lru_cache(maxsize=None)
def _make_spmm():
    mesh = plsc.VectorSubcoreMesh(core_axis_name="c", subcore_axis_name="s",
                                  num_cores=NSC, num_subcores=NT)
    scratch = (
        [pltpu.VMEM_SHARED((NP, CPB), jnp.bfloat16)]
        + [pltpu.VMEM((2, K), jnp.int32) for _ in range(NBUF)]
        + [pltpu.VMEM((K, 32), jnp.bfloat16) for _ in range(NBUF)]
        + [pltpu.VMEM((K, CPB), jnp.bfloat16) for _ in range(NBUF)]
        + [pltpu.SemaphoreType.DMA for _ in range(4 * NBUF)]
    )
    return pl.kernel(
        _spmm_body,
        out_type=[jax.ShapeDtypeStruct((NSC * NP, CPB), jnp.bfloat16),
                  jax.ShapeDtypeStruct((NSC * NP, CPB), jnp.bfloat16)],
        mesh=mesh,
        scratch_types=scratch,
        compiler_params=pltpu.CompilerParams(use_tc_tiling_on_sc=False),
    )


def _spmm2hop(x, edges, vsplat):
    return _make_spmm()(x, edges, vsplat)


def _prep_body(it_ref, h_ref, x0_ref):
    pad = jnp.zeros((NBLK, FB - 1 - U), jnp.float32)
    for c in range(NSC):
        for bl in range(2):
            b = 2 * c + bl
            sl = slice(bl * FB, (bl + 1) * FB)
            x0_ref[c, :, sl] = jnp.concatenate(
                [it_ref[:, b:b + 1], h_ref[b], pad],
                axis=1).astype(jnp.bfloat16)


def _tc_prep(it, h3):
    grid = (N // NBLK,)
    return pl.pallas_call(
        _prep_body,
        grid=grid,
        in_specs=[
            pl.BlockSpec((NBLK, B), lambda i: (i, 0)),
            pl.BlockSpec((B, NBLK, U), lambda i: (0, i, 0)),
        ],
        out_specs=pl.BlockSpec((NSC, NBLK, CPB), lambda i: (0, i, 0)),
        out_shape=jax.ShapeDtypeStruct((NSC, NP, CPB), jnp.bfloat16),
    )(it, h3)


def _gate_body(x0_ref, x1_ref, x2_ref, wg_ref, bg_ref, it_ref, h_ref,
               u_ref, x0c_ref):
    pad = jnp.zeros((NBLK, FB - 1 - U), jnp.float32)
    for c in range(NSC):
        for bl in range(2):
            b = 2 * c + bl
            sl = slice(bl * FB, (bl + 1) * FB)
            z = (jnp.dot(x0_ref[c][:, sl].astype(jnp.float32), wg_ref[0],
                         preferred_element_type=jnp.float32)
                 + jnp.dot(x1_ref[c][:, sl].astype(jnp.float32), wg_ref[1],
                           preferred_element_type=jnp.float32)
                 + jnp.dot(x2_ref[c][:, sl].astype(jnp.float32), wg_ref[2],
                           preferred_element_type=jnp.float32)
                 + bg_ref[...])
            v = jax.nn.sigmoid(z)
            r = v[:, :U]
            u_ref[b] = v[:, U:]
            x0c_ref[c, :, sl] = jnp.concatenate(
                [it_ref[:, b:b + 1], r * h_ref[b], pad],
                axis=1).astype(jnp.bfloat16)


def _tc_gate(x0, x1, x2, wg, bg, it, h3):
    grid = (N // NBLK,)
    xspec = pl.BlockSpec((NSC, NBLK, CPB), lambda i: (0, i, 0))
    return pl.pallas_call(
        _gate_body,
        grid=grid,
        in_specs=[
            xspec, xspec, xspec,
            pl.BlockSpec((3, FB, 2 * U), lambda i: (0, 0, 0)),
            pl.BlockSpec((1, 2 * U), lambda i: (0, 0)),
            pl.BlockSpec((NBLK, B), lambda i: (i, 0)),
            pl.BlockSpec((B, NBLK, U), lambda i: (0, i, 0)),
        ],
        out_specs=[
            pl.BlockSpec((B, NBLK, U), lambda i: (0, i, 0)),
            pl.BlockSpec((NSC, NBLK, CPB), lambda i: (0, i, 0)),
        ],
        out_shape=[
            jax.ShapeDtypeStruct((B, N, U), jnp.float32),
            jax.ShapeDtypeStruct((NSC, NP, CPB), jnp.bfloat16),
        ],
    )(x0, x1, x2, wg, bg, it, h3)


def _cand_body(x0_ref, x1_ref, x2_ref, wc_ref, bc_ref, u_ref, h_ref,
               wp_ref, bp_ref, nh_ref, op_ref):
    for c in range(NSC):
        for bl in range(2):
            b = 2 * c + bl
            sl = slice(bl * FB, (bl + 1) * FB)
            z = (jnp.dot(x0_ref[c][:, sl].astype(jnp.float32), wc_ref[0],
                         preferred_element_type=jnp.float32)
                 + jnp.dot(x1_ref[c][:, sl].astype(jnp.float32), wc_ref[1],
                           preferred_element_type=jnp.float32)
                 + jnp.dot(x2_ref[c][:, sl].astype(jnp.float32), wc_ref[2],
                           preferred_element_type=jnp.float32)
                 + bc_ref[...])
            cc = jnp.tanh(z)
            uu = u_ref[b]
            nh = uu * h_ref[b] + (1.0 - uu) * cc
            nh_ref[b] = nh
            op_ref[:, b:b + 1] = (
                jnp.dot(nh, wp_ref[...], preferred_element_type=jnp.float32)
                + bp_ref[...])


def _tc_cand(x0, x1, x2, wc, bc, u, h3, wp, bp):
    grid = (N // NBLK,)
    xspec = pl.BlockSpec((NSC, NBLK, CPB), lambda i: (0, i, 0))
    uspec = pl.BlockSpec((B, NBLK, U), lambda i: (0, i, 0))
    return pl.pallas_call(
        _cand_body,
        grid=grid,
        in_specs=[
            xspec, xspec, xspec,
            pl.BlockSpec((3, FB, U), lambda i: (0, 0, 0)),
            pl.BlockSpec((1, U), lambda i: (0, 0)),
            uspec, uspec,
            pl.BlockSpec((U, 1), lambda i: (0, 0)),
            pl.BlockSpec((1, 1), lambda i: (0, 0)),
        ],
        out_specs=[
            uspec,
            pl.BlockSpec((NBLK, B), lambda i: (i, 0)),
        ],
        out_shape=[
            jax.ShapeDtypeStruct((B, N, U), jnp.float32),
            jax.ShapeDtypeStruct((N, B), jnp.float32),
        ],
    )(x0, x1, x2, wc, bc, u, h3, wp, bp)


def _fold_cheb(w):
    return jnp.stack([w[0] - w[2], w[1], 2.0 * w[2]])


def kernel(inputs, hidden_state, support_src, support_dst, support_vals,
           W_gate, b_gate, W_cand, b_cand, W_proj, b_proj):
    h3 = hidden_state[0].reshape(B, N, U)
    inputs_t = inputs.T

    npad = NT * EPT - E
    srcp = jnp.concatenate([support_src, jnp.zeros((npad,), jnp.int32)])
    dstp = jnp.concatenate([support_dst, jnp.zeros((npad,), jnp.int32)])
    valp = jnp.concatenate([support_vals, jnp.zeros((npad,), jnp.float32)])
    dst4 = dstp.reshape(1, NT, NCH, 1, K)
    edges = jnp.concatenate([
        jnp.stack([srcp, srcp + NP]).reshape(NSC, NT, NCH, 1, K),
        jnp.concatenate([dst4, dst4], axis=0),
    ], axis=3).reshape(NSC * NT, NCH, 2, K)
    vsplat = jnp.broadcast_to(valp.astype(jnp.bfloat16)[:, None],
                              (NT * EPT, 32)).reshape(NT, NCH, K, 32)

    wg = _fold_cheb(jnp.pad(W_gate.reshape(65, 3, 2 * U).transpose(1, 0, 2),
                            ((0, 0), (0, FB - 65), (0, 0))))
    wc = _fold_cheb(jnp.pad(W_cand.reshape(65, 3, U).transpose(1, 0, 2),
                            ((0, 0), (0, FB - 65), (0, 0))))
    bg = b_gate.reshape(1, 2 * U)
    bc = b_cand.reshape(1, U)
    bp = b_proj.reshape(1, 1)

    x0g = _tc_prep(inputs_t, h3)
    x0g2 = x0g.reshape(NSC * NP, CPB)
    x1g, a2g = _spmm2hop(x0g2, edges, vsplat)
    u, x0c = _tc_gate(x0g, x1g.reshape(NSC, NP, CPB),
                      a2g.reshape(NSC, NP, CPB), wg, bg, inputs_t, h3)
    x0c2 = x0c.reshape(NSC * NP, CPB)
    x1c, a2c = _spmm2hop(x0c2, edges, vsplat)
    newh, outp = _tc_cand(x0c, x1c.reshape(NSC, NP, CPB),
                          a2c.reshape(NSC, NP, CPB), wc, bc, u, h3,
                          W_proj, bp)
    output = outp.T.reshape(B, N)
    return output, jnp.stack([newh.reshape(B, N * U)], axis=0)


# speedup vs baseline: 1.0681x; 1.0681x over previous
import functools

import jax
import jax.numpy as jnp
from jax import lax
from jax.experimental import pallas as pl
from jax.experimental.pallas import tpu as pltpu
from jax.experimental.pallas import tpu_sc as plsc

N = 10000
NP = 10240
E = 160000
U = 64
B = 4
FB = 80
NSC = 2
NT = 16
CPB = 2 * FB
LGB = CPB // 32
K = 160
EPT = 10240
NCH = EPT // K
RPT = NP // NT
NWO = RPT // K
NBUF = 4
NBLK = 2000


def _spmm_body(x_hbm, edge_hbm, vs_hbm, out_hbm, acc, *bufs):
    ebufs = bufs[0:NBUF]
    vbufs = bufs[NBUF:2 * NBUF]
    rowss = bufs[2 * NBUF:3 * NBUF]
    esems = bufs[3 * NBUF:4 * NBUF]
    vsems = bufs[4 * NBUF:5 * NBUF]
    gsems = bufs[5 * NBUF:6 * NBUF]
    ssems = bufs[6 * NBUF:7 * NBUF]
    cid = lax.axis_index("c")
    sid = lax.axis_index("s")
    wid = cid * NT + sid

    def ecopy_start(ch, s):
        pltpu.async_copy(edge_hbm.at[wid, ch], ebufs[s], esems[s])
        pltpu.async_copy(vs_hbm.at[sid, ch], vbufs[s], vsems[s])

    def ecopy_wait(ch, s):
        pltpu.make_async_copy(edge_hbm.at[wid, ch], ebufs[s],
                              esems[s]).wait()
        pltpu.make_async_copy(vs_hbm.at[sid, ch], vbufs[s],
                              vsems[s]).wait()

    def gather_start(s):
        pltpu.async_copy(x_hbm.at[ebufs[s].at[0]], rowss[s], gsems[s])

    def gather_wait(s):
        pltpu.make_async_copy(x_hbm.at[ebufs[s].at[0]], rowss[s],
                              gsems[s]).wait()

    def scatter_start(s):
        pltpu.async_copy(rowss[s], acc.at[ebufs[s].at[1]], ssems[s],
                         add=True)

    def scatter_wait(s):
        pltpu.make_async_copy(rowss[s], acc.at[ebufs[s].at[1]],
                              ssems[s]).wait()

    def scale(s):
        rb = rowss[s]
        vb = vbufs[s]

        @plsc.parallel_loop(0, K, unroll=8)
        def _(e):
            vvb = vb[e, pl.ds(0, 32)]
            for j in range(LGB):
                sl = pl.ds(j * 32, 32)
                rb[e, sl] = rb[e, sl] * vvb

    zero32 = jnp.zeros((32,), jnp.bfloat16)

    def zrow(i, carry):
        for j in range(LGB):
            rowss[0][i, pl.ds(j * 32, 32)] = zero32
        return carry

    lax.fori_loop(0, K, zrow, 0)
    for kk in range(NWO):
        pltpu.sync_copy(rowss[0], acc.at[pl.ds(sid * RPT + kk * K, K)])

    ecopy_start(0, 0)
    ecopy_start(1, 1)
    ecopy_wait(0, 0)
    gather_start(0)
    plsc.subcore_barrier()

    def chunk_iter(ch, t):
        @pl.when(ch + 1 < NCH)
        def _():
            ecopy_wait(ch + 1, (t + 1) % NBUF)
            gather_start((t + 1) % NBUF)

        @pl.when(ch >= 2)
        def _():
            scatter_wait((t + 2) % NBUF)

        @pl.when(ch + 2 < NCH)
        def _():
            ecopy_start(ch + 2, (t + 2) % NBUF)

        gather_wait(t)
        scale(t)
        scatter_start(t)

    def quad(p, carry):
        for t in range(NBUF):
            chunk_iter(NBUF * p + t, t)
        return carry

    lax.fori_loop(0, NCH // NBUF, quad, 0)
    scatter_wait((NCH - 2) % NBUF)
    scatter_wait((NCH - 1) % NBUF)
    plsc.subcore_barrier()

    for kk in range(NWO):
        r0 = sid * RPT + kk * K
        pltpu.sync_copy(acc.at[pl.ds(r0, K)], rowss[kk])
        pltpu.async_copy(rowss[kk], out_hbm.at[pl.ds(cid * NP + r0, K)],
                         gsems[kk])
    for kk in range(NWO):
        r0 = sid * RPT + kk * K
        pltpu.make_async_copy(
            rowss[kk], out_hbm.at[pl.ds(cid * NP + r0, K)],
            gsems[kk]).wait()


@functools.lru_cache(maxsize=None)
def _make_spmm():
    mesh = plsc.VectorSubcoreMesh(core_axis_name="c", subcore_axis_name="s",
                                  num_cores=NSC, num_subcores=NT)
    scratch = (
        [pltpu.VMEM_SHARED((NP, CPB), jnp.bfloat16)]
        + [pltpu.VMEM((2, K), jnp.int32) for _ in range(NBUF)]
        + [pltpu.VMEM((K, 32), jnp.bfloat16) for _ in range(NBUF)]
        + [pltpu.VMEM((K, CPB), jnp.bfloat16) for _ in range(NBUF)]
        + [pltpu.SemaphoreType.DMA for _ in range(4 * NBUF)]
    )
    return pl.kernel(
        _spmm_body,
        out_type=jax.ShapeDtypeStruct((NSC * NP, CPB), jnp.bfloat16),
        mesh=mesh,
        scratch_types=scratch,
        compiler_params=pltpu.CompilerParams(use_tc_tiling_on_sc=False),
    )


def _spmm(x, edges, vsplat):
    return _make_spmm()(x, edges, vsplat)


def _prep_body(it_ref, h_ref, x0_ref):
    pad = jnp.zeros((NBLK, FB - 1 - U), jnp.float32)
    for c in range(NSC):
        for bl in range(2):
            b = 2 * c + bl
            sl = slice(bl * FB, (bl + 1) * FB)
            x0_ref[c, :, sl] = jnp.concatenate(
                [it_ref[:, b:b + 1], h_ref[b], pad],
                axis=1).astype(jnp.bfloat16)


def _tc_prep(it, h3):
    grid = (N // NBLK,)
    return pl.pallas_call(
        _prep_body,
        grid=grid,
        in_specs=[
            pl.BlockSpec((NBLK, B), lambda i: (i, 0)),
            pl.BlockSpec((B, NBLK, U), lambda i: (0, i, 0)),
        ],
        out_specs=pl.BlockSpec((NSC, NBLK, CPB), lambda i: (0, i, 0)),
        out_shape=jax.ShapeDtypeStruct((NSC, NP, CPB), jnp.bfloat16),
    )(it, h3)


def _gate_body(x0_ref, x1_ref, x2_ref, wg_ref, bg_ref, it_ref, h_ref,
               u_ref, x0c_ref):
    pad = jnp.zeros((NBLK, FB - 1 - U), jnp.float32)
    for c in range(NSC):
        for bl in range(2):
            b = 2 * c + bl
            sl = slice(bl * FB, (bl + 1) * FB)
            z = (jnp.dot(x0_ref[c][:, sl].astype(jnp.float32), wg_ref[0],
                         preferred_element_type=jnp.float32)
                 + jnp.dot(x1_ref[c][:, sl].astype(jnp.float32), wg_ref[1],
                           preferred_element_type=jnp.float32)
                 + jnp.dot(x2_ref[c][:, sl].astype(jnp.float32), wg_ref[2],
                           preferred_element_type=jnp.float32)
                 + bg_ref[...])
            v = jax.nn.sigmoid(z)
            r = v[:, :U]
            u_ref[b] = v[:, U:]
            x0c_ref[c, :, sl] = jnp.concatenate(
                [it_ref[:, b:b + 1], r * h_ref[b], pad],
                axis=1).astype(jnp.bfloat16)


def _tc_gate(x0, x1, x2, wg, bg, it, h3):
    grid = (N // NBLK,)
    xspec = pl.BlockSpec((NSC, NBLK, CPB), lambda i: (0, i, 0))
    return pl.pallas_call(
        _gate_body,
        grid=grid,
        in_specs=[
            xspec, xspec, xspec,
            pl.BlockSpec((3, FB, 2 * U), lambda i: (0, 0, 0)),
            pl.BlockSpec((1, 2 * U), lambda i: (0, 0)),
            pl.BlockSpec((NBLK, B), lambda i: (i, 0)),
            pl.BlockSpec((B, NBLK, U), lambda i: (0, i, 0)),
        ],
        out_specs=[
            pl.BlockSpec((B, NBLK, U), lambda i: (0, i, 0)),
            pl.BlockSpec((NSC, NBLK, CPB), lambda i: (0, i, 0)),
        ],
        out_shape=[
            jax.ShapeDtypeStruct((B, N, U), jnp.float32),
            jax.ShapeDtypeStruct((NSC, NP, CPB), jnp.bfloat16),
        ],
    )(x0, x1, x2, wg, bg, it, h3)


def _cand_body(x0_ref, x1_ref, x2_ref, wc_ref, bc_ref, u_ref, h_ref,
               wp_ref, bp_ref, nh_ref, op_ref):
    for c in range(NSC):
        for bl in range(2):
            b = 2 * c + bl
            sl = slice(bl * FB, (bl + 1) * FB)
            z = (jnp.dot(x0_ref[c][:, sl].astype(jnp.float32), wc_ref[0],
                         preferred_element_type=jnp.float32)
                 + jnp.dot(x1_ref[c][:, sl].astype(jnp.float32), wc_ref[1],
                           preferred_element_type=jnp.float32)
                 + jnp.dot(x2_ref[c][:, sl].astype(jnp.float32), wc_ref[2],
                           preferred_element_type=jnp.float32)
                 + bc_ref[...])
            cc = jnp.tanh(z)
            uu = u_ref[b]
            nh = uu * h_ref[b] + (1.0 - uu) * cc
            nh_ref[b] = nh
            op_ref[:, b:b + 1] = (
                jnp.dot(nh, wp_ref[...], preferred_element_type=jnp.float32)
                + bp_ref[...])


def _tc_cand(x0, x1, x2, wc, bc, u, h3, wp, bp):
    grid = (N // NBLK,)
    xspec = pl.BlockSpec((NSC, NBLK, CPB), lambda i: (0, i, 0))
    uspec = pl.BlockSpec((B, NBLK, U), lambda i: (0, i, 0))
    return pl.pallas_call(
        _cand_body,
        grid=grid,
        in_specs=[
            xspec, xspec, xspec,
            pl.BlockSpec((3, FB, U), lambda i: (0, 0, 0)),
            pl.BlockSpec((1, U), lambda i: (0, 0)),
            uspec, uspec,
            pl.BlockSpec((U, 1), lambda i: (0, 0)),
            pl.BlockSpec((1, 1), lambda i: (0, 0)),
        ],
        out_specs=[
            uspec,
            pl.BlockSpec((NBLK, B), lambda i: (i, 0)),
        ],
        out_shape=[
            jax.ShapeDtypeStruct((B, N, U), jnp.float32),
            jax.ShapeDtypeStruct((N, B), jnp.float32),
        ],
    )(x0, x1, x2, wc, bc, u, h3, wp, bp)


def _fold_cheb(w):
    return jnp.stack([w[0] - w[2], w[1], 2.0 * w[2]])


def kernel(inputs, hidden_state, support_src, support_dst, support_vals,
           W_gate, b_gate, W_cand, b_cand, W_proj, b_proj):
    h3 = hidden_state[0].reshape(B, N, U)
    inputs_t = inputs.T

    npad = NT * EPT - E
    srcp = jnp.concatenate([support_src, jnp.zeros((npad,), jnp.int32)])
    dstp = jnp.concatenate([support_dst, jnp.zeros((npad,), jnp.int32)])
    valp = jnp.concatenate([support_vals, jnp.zeros((npad,), jnp.float32)])
    dst4 = dstp.reshape(1, NT, NCH, 1, K)
    edges = jnp.concatenate([
        jnp.stack([srcp, srcp + NP]).reshape(NSC, NT, NCH, 1, K),
        jnp.concatenate([dst4, dst4], axis=0),
    ], axis=3).reshape(NSC * NT, NCH, 2, K)
    vsplat = jnp.broadcast_to(valp.astype(jnp.bfloat16)[:, None],
                              (NT * EPT, 32)).reshape(NT, NCH, K, 32)

    wg = _fold_cheb(jnp.pad(W_gate.reshape(65, 3, 2 * U).transpose(1, 0, 2),
                            ((0, 0), (0, FB - 65), (0, 0))))
    wc = _fold_cheb(jnp.pad(W_cand.reshape(65, 3, U).transpose(1, 0, 2),
                            ((0, 0), (0, FB - 65), (0, 0))))
    bg = b_gate.reshape(1, 2 * U)
    bc = b_cand.reshape(1, U)
    bp = b_proj.reshape(1, 1)

    x0g = _tc_prep(inputs_t, h3)
    x0g2 = x0g.reshape(NSC * NP, CPB)
    x1g = _spmm(x0g2, edges, vsplat)
    a2g = _spmm(x1g, edges, vsplat)
    u, x0c = _tc_gate(x0g, x1g.reshape(NSC, NP, CPB),
                      a2g.reshape(NSC, NP, CPB), wg, bg, inputs_t, h3)
    x0c2 = x0c.reshape(NSC * NP, CPB)
    x1c = _spmm(x0c2, edges, vsplat)
    a2c = _spmm(x1c, edges, vsplat)
    newh, outp = _tc_cand(x0c, x1c.reshape(NSC, NP, CPB),
                          a2c.reshape(NSC, NP, CPB), wc, bc, u, h3,
                          W_proj, bp)
    output = outp.T.reshape(B, N)
    return output, jnp.stack([newh.reshape(B, N * U)], axis=0)
